# baseline (speedup 1.0000x reference)
import functools

import jax
import jax.numpy as jnp
from jax import lax
from jax.experimental import pallas as pl
from jax.experimental.pallas import tpu as pltpu
from jax.experimental.pallas import tpu_sc as plsc

N = 4096
D = 1024
OUT = 512
E = 8

B = 256
NB = N // B
T = NB + E - 1

NC = 2
NS = 16
NW = NC * NS
TPW = N // NW
L = 16
NV = N // L
CH = 32
NCH = TPW // CH


def _lane_iota():
    return lax.iota(jnp.int32, L)


def _bcast_lane(vec, e):
    return jnp.sum(jnp.where(_lane_iota() == e, vec, 0))


def _sc_dispatch_body(tid_hbm, x_hbm, xs_hbm, rank_hbm, sched_hbm,
                      tid_v, pre8_v, rank_v, rparts, rows, sems, sched_v):
    wid = lax.axis_index("s") * NC + lax.axis_index("c")
    base = wid * TPW

    pltpu.sync_copy(tid_hbm, tid_v)

    my_first_vreg = wid * (TPW // L)
    zeros8 = tuple(jnp.zeros((L,), jnp.int32) for _ in range(E))

    def count_step(i, accs):
        @pl.when(i == my_first_vreg)
        def _snap():
            for e in range(E):
                pre8_v[e] = accs[e]
        tv = tid_v[pl.ds(i * L, L)]
        return tuple(accs[e] + jnp.where(tv == e, 1, 0) for e in range(E))

    accs = lax.fori_loop(0, NV, count_step, zeros8, unroll=4)

    totals = jnp.zeros((L,), jnp.int32)
    pre = jnp.zeros((L,), jnp.int32)
    lane = _lane_iota()
    for e in range(E):
        totals = totals + jnp.where(lane == e, jnp.sum(accs[e]), 0)
        pre = pre + jnp.where(lane == e, jnp.sum(pre8_v[e]), 0)
    ends = plsc.cumsum(totals)
    offs = ends - totals
    base_vec = offs + pre

    for v in range(TPW // L):
        tv = tid_v[pl.ds((my_first_vreg + v) * L, L)]
        rv = jnp.zeros((L,), jnp.int32)
        for e in range(E):
            mask = tv == e
            ind = jnp.where(mask, 1, 0).astype(jnp.int32)
            cs = plsc.cumsum(ind)
            cnt_e = jnp.max(cs)
            be = _bcast_lane(base_vec, e)
            rv = jnp.where(mask, be + cs - 1, rv)
            base_vec = base_vec + jnp.where(lane == e, cnt_e, 0)
        part = rparts[v // (CH // L)]
        part[pl.ds((v % (CH // L)) * L, L)] = rv
        rank_v[pl.ds(v * L, L)] = rv

    pltpu.sync_copy(rank_v, rank_hbm.at[pl.ds(base, TPW)])

    loads = [None] * NCH
    scats = [None] * NCH
    loads[0] = pltpu.async_copy(
        x_hbm.at[pl.ds(base, CH)], rows[0], sems[0])
    for c in range(NCH):
        loads[c].wait()
        if c + 1 < NCH:
            if c >= 1:
                scats[c - 1].wait()
            loads[c + 1] = pltpu.async_copy(
                x_hbm.at[pl.ds(base + (c + 1) * CH, CH)],
                rows[(c + 1) % 2], sems[(c + 1) % 2])
        scats[c] = pltpu.async_copy(rows[c % 2], xs_hbm.at[rparts[c]],
                                    sems[2 + (c % 2)])
    scats[NCH - 2].wait()
    scats[NCH - 1].wait()

    @pl.when(wid == 0)
    def _sched():
        fb = offs // B
        lb = (ends + (B - 1)) // B - 1
        tpe = jnp.where(totals > 0, lb - fb + 1, 0)
        ts_incl = plsc.cumsum(tpe)
        ts_excl = ts_incl - tpe
        total_tiles = _bcast_lane(ts_incl, E - 1)
        for half in range(2):
            t_vec = lane + half * L
            e_of_t = jnp.zeros((L,), jnp.int32)
            for e in range(E):
                s_e = _bcast_lane(ts_incl, e)
                e_of_t = e_of_t + jnp.where(t_vec >= s_e, 1, 0)
            e_of_t = jnp.minimum(e_of_t, E - 1)
            blk = jnp.zeros((L,), jnp.int32)
            seg_s = jnp.zeros((L,), jnp.int32)
            seg_e = jnp.zeros((L,), jnp.int32)
            for e in range(E):
                sel = e_of_t == e
                blk = jnp.where(
                    sel, _bcast_lane(fb, e) + t_vec - _bcast_lane(ts_excl, e),
                    blk)
                seg_s = jnp.where(sel, _bcast_lane(offs, e), seg_s)
                seg_e = jnp.where(sel, _bcast_lane(ends, e), seg_e)
            valid = jnp.where(t_vec < total_tiles, 1, 0).astype(jnp.int32)
            blk = jnp.where(valid == 1, blk, NB - 1)
            sched_v[0, pl.ds(half * L, L)] = blk
            sched_v[1, pl.ds(half * L, L)] = e_of_t
            sched_v[2, pl.ds(half * L, L)] = valid
            sched_v[3, pl.ds(half * L, L)] = seg_s
            sched_v[4, pl.ds(half * L, L)] = seg_e
        pltpu.sync_copy(sched_v, sched_hbm)


_sc_dispatch = functools.partial(
    pl.kernel,
    mesh=plsc.VectorSubcoreMesh(core_axis_name="c", subcore_axis_name="s"),
    compiler_params=pltpu.CompilerParams(needs_layout_passes=False),
    out_type=(
        jax.ShapeDtypeStruct((N, D), jnp.float32),
        jax.ShapeDtypeStruct((N,), jnp.int32),
        jax.ShapeDtypeStruct((5, NW), jnp.int32),
    ),
    scratch_types=[
        pltpu.VMEM((N,), jnp.int32),
        pltpu.VMEM((E, L), jnp.int32),
        pltpu.VMEM((TPW,), jnp.int32),
        [pltpu.VMEM((CH,), jnp.int32) for _ in range(NCH)],
        [pltpu.VMEM((CH, D), jnp.float32) for _ in range(2)],
        [pltpu.SemaphoreType.DMA for _ in range(4)],
        pltpu.VMEM((5, NW), jnp.int32),
    ],
)(_sc_dispatch_body)


OUTP = OUT + E


def _grouped_body(s_ref, x_ref, w_ref, b_ref, out_ref, wbf_ref):
    t = pl.program_id(0)
    cur_b = s_ref[0, t]
    prev_b = s_ref[0, jnp.maximum(t - 1, 0)]
    first = jnp.logical_or(t == 0, cur_b != prev_b)

    @pl.when(first)
    def _init():
        out_ref[...] = jnp.zeros_like(out_ref)

    new_w = jnp.logical_or(t == 0, s_ref[1, t] != s_ref[1, jnp.maximum(t - 1, 0)])

    @pl.when(new_w)
    def _cast_w():
        wbf_ref[...] = w_ref[0].astype(jnp.bfloat16)

    @pl.when(s_ref[2, t] == 1)
    def _acc():
        e = s_ref[1, t]
        row_ids = cur_b * B + jax.lax.broadcasted_iota(jnp.int32, (B, 1), 0)
        mask = jnp.logical_and(row_ids >= s_ref[3, t], row_ids < s_ref[4, t])
        acc = jnp.dot(x_ref[...].astype(jnp.bfloat16), wbf_ref[...],
                      preferred_element_type=jnp.float32)
        acc = acc + b_ref[0, 0][None, :]
        out_ref[0, :, :OUT] += jnp.where(mask, acc, 0.0)
        tail = (jax.lax.broadcasted_iota(jnp.int32, (B, E), 1) == e
                ).astype(jnp.float32)
        out_ref[0, :, OUT:OUT + E] += jnp.where(mask, tail, 0.0)


def kernel(x, type_ids, W, b):
    tid = type_ids.astype(jnp.int32)
    b3 = b.reshape(E, 1, OUT)

    x_sorted, rank, sched = _sc_dispatch(tid, x)

    grid_spec = pltpu.PrefetchScalarGridSpec(
        num_scalar_prefetch=1,
        grid=(T,),
        in_specs=[
            pl.BlockSpec((B, D), lambda t, s: (s[0, t], 0)),
            pl.BlockSpec((1, D, OUT), lambda t, s: (s[1, t], 0, 0)),
            pl.BlockSpec((1, 1, OUT), lambda t, s: (s[1, t], 0, 0)),
        ],
        out_specs=pl.BlockSpec((1, B, OUTP), lambda t, s: (s[0, t], 0, 0)),
        scratch_shapes=[pltpu.VMEM((D, OUT), jnp.bfloat16)],
    )
    rows_sorted = pl.pallas_call(
        _grouped_body,
        grid_spec=grid_spec,
        out_shape=jax.ShapeDtypeStruct((NB, B, OUTP), jnp.float32),
    )(sched, x_sorted, W, b3)

    gathered = rows_sorted.reshape(N, 1, OUTP)[rank]
    scale = jnp.where(sched[2, 0] >= 0, 1.0, 2.0).astype(jnp.float32)
    return gathered * scale


# speedup vs baseline: 1.1298x; 1.1298x over previous
import functools

import jax
import jax.numpy as jnp
from jax import lax
from jax.experimental import pallas as pl
from jax.experimental.pallas import tpu as pltpu
from jax.experimental.pallas import tpu_sc as plsc

N = 4096
D = 1024
OUT = 512
E = 8

B = 512
NB = N // B
T = NB + E - 1

NC = 2
NS = 16
NW = NC * NS
TPW = N // NW
L = 16
NV = N // L
CH = 32
NCH = TPW // CH


def _lane_iota():
    return lax.iota(jnp.int32, L)


def _bcast_lane(vec, e):
    return jnp.sum(jnp.where(_lane_iota() == e, vec, 0))


def _sc_dispatch_body(tid_hbm, x_hbm, xs_hbm, rank_hbm, sched_hbm,
                      tid_v, pre8_v, rank_v, rparts, rows, sems, sched_v):
    wid = lax.axis_index("s") * NC + lax.axis_index("c")
    base = wid * TPW

    pltpu.sync_copy(tid_hbm, tid_v)

    my_first_vreg = wid * (TPW // L)
    zeros8 = tuple(jnp.zeros((L,), jnp.int32) for _ in range(E))

    def count_step(i, accs):
        @pl.when(i == my_first_vreg)
        def _snap():
            for e in range(E):
                pre8_v[e] = accs[e]
        tv = tid_v[pl.ds(i * L, L)]
        return tuple(accs[e] + jnp.where(tv == e, 1, 0) for e in range(E))

    accs = lax.fori_loop(0, NV, count_step, zeros8, unroll=4)

    totals = jnp.zeros((L,), jnp.int32)
    pre = jnp.zeros((L,), jnp.int32)
    lane = _lane_iota()
    for e in range(E):
        totals = totals + jnp.where(lane == e, jnp.sum(accs[e]), 0)
        pre = pre + jnp.where(lane == e, jnp.sum(pre8_v[e]), 0)
    ends = plsc.cumsum(totals)
    offs = ends - totals
    base_vec = offs + pre

    for v in range(TPW // L):
        tv = tid_v[pl.ds((my_first_vreg + v) * L, L)]
        rv = jnp.zeros((L,), jnp.int32)
        for e in range(E):
            mask = tv == e
            ind = jnp.where(mask, 1, 0).astype(jnp.int32)
            cs = plsc.cumsum(ind)
            cnt_e = jnp.max(cs)
            be = _bcast_lane(base_vec, e)
            rv = jnp.where(mask, be + cs - 1, rv)
            base_vec = base_vec + jnp.where(lane == e, cnt_e, 0)
        part = rparts[v // (CH // L)]
        part[pl.ds((v % (CH // L)) * L, L)] = rv
        rank_v[pl.ds(v * L, L)] = rv

    pltpu.sync_copy(rank_v, rank_hbm.at[pl.ds(base, TPW)])

    loads = [None] * NCH
    scats = [None] * NCH
    loads[0] = pltpu.async_copy(
        x_hbm.at[pl.ds(base, CH)], rows[0], sems[0])
    for c in range(NCH):
        loads[c].wait()
        if c + 1 < NCH:
            if c >= 1:
                scats[c - 1].wait()
            loads[c + 1] = pltpu.async_copy(
                x_hbm.at[pl.ds(base + (c + 1) * CH, CH)],
                rows[(c + 1) % 2], sems[(c + 1) % 2])
        scats[c] = pltpu.async_copy(rows[c % 2], xs_hbm.at[rparts[c]],
                                    sems[2 + (c % 2)])
    scats[NCH - 2].wait()
    scats[NCH - 1].wait()

    @pl.when(wid == 0)
    def _sched():
        fb = offs // B
        lb = (ends + (B - 1)) // B - 1
        tpe = jnp.where(totals > 0, lb - fb + 1, 0)
        ts_incl = plsc.cumsum(tpe)
        ts_excl = ts_incl - tpe
        total_tiles = _bcast_lane(ts_incl, E - 1)
        for half in range(2):
            t_vec = lane + half * L
            e_of_t = jnp.zeros((L,), jnp.int32)
            for e in range(E):
                s_e = _bcast_lane(ts_incl, e)
                e_of_t = e_of_t + jnp.where(t_vec >= s_e, 1, 0)
            e_of_t = jnp.minimum(e_of_t, E - 1)
            blk = jnp.zeros((L,), jnp.int32)
            seg_s = jnp.zeros((L,), jnp.int32)
            seg_e = jnp.zeros((L,), jnp.int32)
            for e in range(E):
                sel = e_of_t == e
                blk = jnp.where(
                    sel, _bcast_lane(fb, e) + t_vec - _bcast_lane(ts_excl, e),
                    blk)
                seg_s = jnp.where(sel, _bcast_lane(offs, e), seg_s)
                seg_e = jnp.where(sel, _bcast_lane(ends, e), seg_e)
            valid = jnp.where(t_vec < total_tiles, 1, 0).astype(jnp.int32)
            blk = jnp.where(valid == 1, blk, NB - 1)
            sched_v[0, pl.ds(half * L, L)] = blk
            sched_v[1, pl.ds(half * L, L)] = e_of_t
            sched_v[2, pl.ds(half * L, L)] = valid
            sched_v[3, pl.ds(half * L, L)] = seg_s
            sched_v[4, pl.ds(half * L, L)] = seg_e
        pltpu.sync_copy(sched_v, sched_hbm)


_sc_dispatch = functools.partial(
    pl.kernel,
    mesh=plsc.VectorSubcoreMesh(core_axis_name="c", subcore_axis_name="s"),
    compiler_params=pltpu.CompilerParams(needs_layout_passes=False),
    out_type=(
        jax.ShapeDtypeStruct((N, D), jnp.float32),
        jax.ShapeDtypeStruct((N,), jnp.int32),
        jax.ShapeDtypeStruct((5, NW), jnp.int32),
    ),
    scratch_types=[
        pltpu.VMEM((N,), jnp.int32),
        pltpu.VMEM((E, L), jnp.int32),
        pltpu.VMEM((TPW,), jnp.int32),
        [pltpu.VMEM((CH,), jnp.int32) for _ in range(NCH)],
        [pltpu.VMEM((CH, D), jnp.float32) for _ in range(2)],
        [pltpu.SemaphoreType.DMA for _ in range(4)],
        pltpu.VMEM((5, NW), jnp.int32),
    ],
)(_sc_dispatch_body)


OUTP = OUT + E


def _grouped_body(s_ref, x_ref, w_ref, b_ref, out_ref, wbf_ref):
    t = pl.program_id(0)
    cur_b = s_ref[0, t]
    prev_b = s_ref[0, jnp.maximum(t - 1, 0)]
    first = jnp.logical_or(t == 0, cur_b != prev_b)

    @pl.when(first)
    def _init():
        out_ref[...] = jnp.zeros_like(out_ref)

    new_w = jnp.logical_or(t == 0, s_ref[1, t] != s_ref[1, jnp.maximum(t - 1, 0)])

    @pl.when(new_w)
    def _cast_w():
        wbf_ref[...] = w_ref[0].astype(jnp.bfloat16)

    @pl.when(s_ref[2, t] == 1)
    def _acc():
        e = s_ref[1, t]
        row_ids = cur_b * B + jax.lax.broadcasted_iota(jnp.int32, (B, 1), 0)
        mask = jnp.logical_and(row_ids >= s_ref[3, t], row_ids < s_ref[4, t])
        acc = jnp.dot(x_ref[...].astype(jnp.bfloat16), wbf_ref[...],
                      preferred_element_type=jnp.float32)
        acc = acc + b_ref[0, 0][None, :]
        out_ref[0, :, :OUT] += jnp.where(mask, acc, 0.0)
        tail = (jax.lax.broadcasted_iota(jnp.int32, (B, E), 1) == e
                ).astype(jnp.float32)
        out_ref[0, :, OUT:OUT + E] += jnp.where(mask, tail, 0.0)


def kernel(x, type_ids, W, b):
    tid = type_ids.astype(jnp.int32)
    b3 = b.reshape(E, 1, OUT)

    x_sorted, rank, sched = _sc_dispatch(tid, x)

    grid_spec = pltpu.PrefetchScalarGridSpec(
        num_scalar_prefetch=1,
        grid=(T,),
        in_specs=[
            pl.BlockSpec((B, D), lambda t, s: (s[0, t], 0)),
            pl.BlockSpec((1, D, OUT), lambda t, s: (s[1, t], 0, 0)),
            pl.BlockSpec((1, 1, OUT), lambda t, s: (s[1, t], 0, 0)),
        ],
        out_specs=pl.BlockSpec((1, B, OUTP), lambda t, s: (s[0, t], 0, 0)),
        scratch_shapes=[pltpu.VMEM((D, OUT), jnp.bfloat16)],
    )
    rows_sorted = pl.pallas_call(
        _grouped_body,
        grid_spec=grid_spec,
        out_shape=jax.ShapeDtypeStruct((NB, B, OUTP), jnp.float32),
    )(sched, x_sorted, W, b3)

    return rows_sorted.reshape(N, 1, OUTP)[rank]
